# baseline (speedup 1.0000x reference)
import functools

import jax
import jax.numpy as jnp
from jax import lax
from jax.experimental import pallas as pl
from jax.experimental.pallas import tpu as pltpu
from jax.experimental.pallas import tpu_sc as plsc

_NT = 4096
_NS = 768
_ROWS = 256
_WORKERS = 32
_PER_W = _NS // _WORKERS
_BIG = 3.0e38
_OBS_BUF = _NT + 2048


def _tc_prep_body(x_ref, y_ref, syn_ref, obs_ref):
    xv = x_ref[...]
    yv = y_ref[...]
    mind = jnp.minimum(
        jnp.min(xv, axis=0, keepdims=True),
        jnp.min(yv, axis=0, keepdims=True),
    )
    row = lax.broadcasted_iota(jnp.int32, (_NT, _ROWS), 0)
    valid_t = row < (_NT - 1)
    zrow = jnp.zeros((1, _ROWS), jnp.float32)
    ch = 128
    ia = lax.broadcasted_iota(jnp.int32, (ch, ch), 0)
    ib = lax.broadcasted_iota(jnp.int32, (ch, ch), 1)
    tri = (ia <= ib).astype(jnp.float32)
    lane = lax.broadcasted_iota(jnp.int32, (_ROWS, _NT), 1)
    valid = lane < (_NT - 1)
    for v, out, padval in ((xv, syn_ref, 0.0), (yv, obs_ref, _BIG)):
        s = v - mind
        s_next = jnp.concatenate([s[1:, :], zrow], axis=0)
        tz = jnp.where(valid_t, (s + s_next) * 0.5, 0.0)
        carry = jnp.zeros((_ROWS, 1), jnp.float32)
        pieces = []
        for t in range(_NT // ch):
            blk = tz[t * ch : (t + 1) * ch, :]
            cs = (
                lax.dot_general(
                    blk,
                    tri,
                    dimension_numbers=(((0,), (0,)), ((), ())),
                    preferred_element_type=jnp.float32,
                )
                + carry
            )
            carry = cs[:, ch - 1 : ch]
            pieces.append(cs)
        c = jnp.concatenate(pieces, axis=1)
        total = jnp.sum(c, axis=1, keepdims=True) - carry
        out[...] = jnp.where(valid, c / total, padval)


def _tc_prep(x2d, y2d):
    grid = _NS // _ROWS
    in_spec = pl.BlockSpec((_NT, _ROWS), lambda i: (0, i))
    out_spec = pl.BlockSpec((_ROWS, _NT), lambda i: (i, 0))
    return pl.pallas_call(
        _tc_prep_body,
        grid=(grid,),
        in_specs=[in_spec, in_spec],
        out_specs=[out_spec, out_spec],
        out_shape=[
            jax.ShapeDtypeStruct((_NS, _NT), jnp.float32),
            jax.ShapeDtypeStruct((_NS, _NT), jnp.float32),
        ],
    )(x2d, y2d)


_UNROLL = 4


def _search_col(syn_v, obs_v, cpos_v, acc):
    lane = lax.iota(jnp.int32, 16)

    def coarse_body(j, c):
        qidx = jnp.minimum(j * 256 + lane * 16 + 15, _NT - 2)
        q = plsc.load_gather(syn_v, [qidx])
        pos = jnp.zeros((16,), jnp.int32)
        d = _NT // 2
        while d >= 1:
            probe = pos + (d - 1)
            v = plsc.load_gather(obs_v, [probe])
            pos = jnp.where(v < q, probe + 1, pos)
            d //= 2
        cpos_v[pl.ds(j * 16 + 1, 16)] = pos
        return c

    lax.fori_loop(0, 256 // 16, coarse_body, 0)

    def group_body(g, a):
        c0 = g * _UNROLL
        cv = cpos_v[pl.ds(c0, 16)]
        los = [cv[u] for u in range(_UNROLL)]
        his = [cv[u + 1] for u in range(_UNROLL)]
        ws = [h - l for h, l in zip(his, los)]
        wmax = jnp.maximum(
            jnp.maximum(ws[0], ws[1]), jnp.maximum(ws[2], ws[3])
        )
        p = wmax
        for sh in (1, 2, 4, 8):
            p = p | (p >> sh)
        d0 = (p + 1) >> 1
        qs = [syn_v[pl.ds((c0 + u) * 16, 16)] for u in range(_UNROLL)]
        poss = [jnp.zeros((16,), jnp.int32) + l for l in los]

        def wcond(s):
            return s[0] >= 1

        def wbody(s):
            d = s[0]
            ps = list(s[1:])
            dm1 = d - 1
            for u in range(_UNROLL):
                probe = ps[u] + dm1
                v = plsc.load_gather(obs_v, [probe])
                ps[u] = jnp.where(v < qs[u], probe + 1, ps[u])
            return (d >> 1, *ps)

        fin = lax.while_loop(wcond, wbody, (d0, *poss))
        for u in range(_UNROLL):
            diff = ((c0 + u) * 16 + 1 + lane - fin[1 + u]).astype(jnp.float32)
            a = a + diff * diff * qs[u]
        return a

    return lax.fori_loop(0, _NT // (16 * _UNROLL), group_body, acc)


def _sc_search_body(
    syn_hbm, obs_hbm, out_hbm, syn0, obs0, syn1, obs1, cpos_v, acc_v,
    sem0, sem1
):
    info = plsc.get_sparse_core_info()
    nc = info.num_cores
    wid = lax.axis_index("s") * nc + lax.axis_index("c")
    base = wid * _PER_W
    last = base + _PER_W - 1

    big = jnp.full((16,), _BIG, jnp.float32)

    def fill_body(i, c):
        obs0[pl.ds(_NT + i * 16, 16)] = big
        obs1[pl.ds(_NT + i * 16, 16)] = big
        return c

    lax.fori_loop(0, (_OBS_BUF - _NT) // 16, fill_body, 0)
    cpos_v[pl.ds(0, 16)] = jnp.zeros((16,), jnp.int32)

    pltpu.sync_copy(syn_hbm.at[base], syn0)
    pltpu.sync_copy(obs_hbm.at[base], obs0.at[pl.ds(0, _NT)])

    def col2_body(jj, acc):
        c0 = base + jj * 2
        nxt = jnp.minimum(c0 + 1, last)
        h1 = pltpu.async_copy(syn_hbm.at[nxt], syn1, sem1)
        h2 = pltpu.async_copy(obs_hbm.at[nxt], obs1.at[pl.ds(0, _NT)], sem1)
        acc = _search_col(syn0, obs0, cpos_v, acc)
        h1.wait()
        h2.wait()
        nxt2 = jnp.minimum(c0 + 2, last)
        h3 = pltpu.async_copy(syn_hbm.at[nxt2], syn0, sem0)
        h4 = pltpu.async_copy(obs_hbm.at[nxt2], obs0.at[pl.ds(0, _NT)], sem0)
        acc = _search_col(syn1, obs1, cpos_v, acc)
        h3.wait()
        h4.wait()
        return acc

    acc = lax.fori_loop(
        0, _PER_W // 2, col2_body, jnp.zeros((16,), jnp.float32)
    )
    acc_v[...] = acc
    pltpu.sync_copy(acc_v, out_hbm.at[wid])


def _sc_search(syn, obs):
    mesh = plsc.VectorSubcoreMesh(core_axis_name="c", subcore_axis_name="s")
    kern = functools.partial(
        pl.kernel,
        out_type=jax.ShapeDtypeStruct((_WORKERS, 16), jnp.float32),
        mesh=mesh,
        scratch_types=[
            pltpu.VMEM((_NT,), jnp.float32),
            pltpu.VMEM((_OBS_BUF,), jnp.float32),
            pltpu.VMEM((_NT,), jnp.float32),
            pltpu.VMEM((_OBS_BUF,), jnp.float32),
            pltpu.VMEM((272,), jnp.int32),
            pltpu.VMEM((16,), jnp.float32),
            pltpu.SemaphoreType.DMA,
            pltpu.SemaphoreType.DMA,
        ],
        compiler_params=pltpu.CompilerParams(needs_layout_passes=False),
    )(_sc_search_body)
    return kern(syn, obs)


def kernel(x, y):
    syn, obs = _tc_prep(x.reshape(_NT, -1), y.reshape(_NT, -1))
    part = _sc_search(syn, obs)
    return jnp.sum(part)


# speedup vs baseline: 1.3211x; 1.3211x over previous
import functools

import jax
import jax.numpy as jnp
from jax import lax
from jax.experimental import pallas as pl
from jax.experimental.pallas import tpu as pltpu
from jax.experimental.pallas import tpu_sc as plsc

_NT = 4096
_NS = 768
_ROWS = 256
_WORKERS = 32
_PER_W = _NS // _WORKERS


def _tc_prep_body(x_ref, y_ref, syn_ref, obs_ref):
    xv = x_ref[...]
    yv = y_ref[...]
    mind = jnp.minimum(
        jnp.min(xv, axis=0, keepdims=True),
        jnp.min(yv, axis=0, keepdims=True),
    )
    row = lax.broadcasted_iota(jnp.int32, (_NT, _ROWS), 0)
    valid_t = row < (_NT - 1)
    zrow = jnp.zeros((1, _ROWS), jnp.float32)
    ch = 128
    ia = lax.broadcasted_iota(jnp.int32, (ch, ch), 0)
    ib = lax.broadcasted_iota(jnp.int32, (ch, ch), 1)
    tri = (ia <= ib).astype(jnp.float32)
    lane = lax.broadcasted_iota(jnp.int32, (_ROWS, _NT), 1)
    valid = lane < (_NT - 1)
    for v, out in ((xv, syn_ref), (yv, obs_ref)):
        s = v - mind
        s_next = jnp.concatenate([s[1:, :], zrow], axis=0)
        tz = jnp.where(valid_t, (s + s_next) * 0.5, 0.0)
        carry = jnp.zeros((_ROWS, 1), jnp.float32)
        pieces = []
        for t in range(_NT // ch):
            blk = tz[t * ch : (t + 1) * ch, :]
            cs = (
                lax.dot_general(
                    blk,
                    tri,
                    dimension_numbers=(((0,), (0,)), ((), ())),
                    preferred_element_type=jnp.float32,
                )
                + carry
            )
            carry = cs[:, ch - 1 : ch]
            pieces.append(cs)
        c = jnp.concatenate(pieces, axis=1)
        total = jnp.sum(c, axis=1, keepdims=True) - carry
        out[...] = jnp.where(valid, c / total, 0.0)


def _tc_prep(x2d, y2d):
    grid = _NS // _ROWS
    in_spec = pl.BlockSpec((_NT, _ROWS), lambda i: (0, i))
    out_spec = pl.BlockSpec((_ROWS, _NT), lambda i: (i, 0))
    return pl.pallas_call(
        _tc_prep_body,
        grid=(grid,),
        in_specs=[in_spec, in_spec],
        out_specs=[out_spec, out_spec],
        out_shape=[
            jax.ShapeDtypeStruct((_NS, _NT), jnp.float32),
            jax.ShapeDtypeStruct((_NS, _NT), jnp.float32),
        ],
    )(x2d, y2d)


_UNROLL = 8


def _search_col(syn_v, obs_v, acc):
    lane = lax.iota(jnp.int32, 16)

    def chunk_body(k, a):
        for u in range(_UNROLL):
            ck = k * _UNROLL + u
            q = syn_v[pl.ds(ck * 16, 16)]
            pos = jnp.zeros((16,), jnp.int32)
            d = _NT // 2
            while d >= 1:
                probe = pos + (d - 1)
                v = plsc.load_gather(obs_v, [probe])
                pos = jnp.where(v < q, probe + 1, pos)
                d //= 2
            diff = (ck * 16 + 1 + lane - pos).astype(jnp.float32)
            a = a + diff * diff * q
        return a

    return lax.fori_loop(0, _NT // (16 * _UNROLL), chunk_body, acc)


def _sc_search_body(
    syn_hbm, obs_hbm, out_hbm, syn0, obs0, syn1, obs1, acc_v, sem0, sem1
):
    info = plsc.get_sparse_core_info()
    nc = info.num_cores
    wid = lax.axis_index("s") * nc + lax.axis_index("c")
    base = wid * _PER_W
    last = base + _PER_W - 1

    pltpu.sync_copy(syn_hbm.at[base], syn0)
    pltpu.sync_copy(obs_hbm.at[base], obs0)

    def col2_body(jj, acc):
        c0 = base + jj * 2
        nxt = jnp.minimum(c0 + 1, last)
        h1 = pltpu.async_copy(syn_hbm.at[nxt], syn1, sem1)
        h2 = pltpu.async_copy(obs_hbm.at[nxt], obs1, sem1)
        acc = _search_col(syn0, obs0, acc)
        h1.wait()
        h2.wait()
        nxt2 = jnp.minimum(c0 + 2, last)
        h3 = pltpu.async_copy(syn_hbm.at[nxt2], syn0, sem0)
        h4 = pltpu.async_copy(obs_hbm.at[nxt2], obs0, sem0)
        acc = _search_col(syn1, obs1, acc)
        h3.wait()
        h4.wait()
        return acc

    acc = lax.fori_loop(
        0, _PER_W // 2, col2_body, jnp.zeros((16,), jnp.float32)
    )
    acc_v[...] = acc
    pltpu.sync_copy(acc_v, out_hbm.at[wid])


def _sc_search(syn, obs):
    mesh = plsc.VectorSubcoreMesh(core_axis_name="c", subcore_axis_name="s")
    kern = functools.partial(
        pl.kernel,
        out_type=jax.ShapeDtypeStruct((_WORKERS, 16), jnp.float32),
        mesh=mesh,
        scratch_types=[
            pltpu.VMEM((_NT,), jnp.float32),
            pltpu.VMEM((_NT,), jnp.float32),
            pltpu.VMEM((_NT,), jnp.float32),
            pltpu.VMEM((_NT,), jnp.float32),
            pltpu.VMEM((16,), jnp.float32),
            pltpu.SemaphoreType.DMA,
            pltpu.SemaphoreType.DMA,
        ],
        compiler_params=pltpu.CompilerParams(needs_layout_passes=False),
    )(_sc_search_body)
    return kern(syn, obs)


def kernel(x, y):
    syn, obs = _tc_prep(x.reshape(_NT, -1), y.reshape(_NT, -1))
    part = _sc_search(syn, obs)
    return jnp.sum(part)


# speedup vs baseline: 1.3234x; 1.0017x over previous
import functools

import jax
import jax.numpy as jnp
from jax import lax
from jax.experimental import pallas as pl
from jax.experimental.pallas import tpu as pltpu
from jax.experimental.pallas import tpu_sc as plsc

_NT = 4096
_NS = 768
_ROWS = 256
_WORKERS = 32
_PER_W = _NS // _WORKERS


def _tc_prep_body(x_ref, y_ref, syn_ref, obs_ref):
    xv = x_ref[...]
    yv = y_ref[...]
    mind = jnp.minimum(
        jnp.min(xv, axis=0, keepdims=True),
        jnp.min(yv, axis=0, keepdims=True),
    )
    row = lax.broadcasted_iota(jnp.int32, (_NT, _ROWS), 0)
    valid_t = row < (_NT - 1)
    zrow = jnp.zeros((1, _ROWS), jnp.float32)
    ch = 128
    ia = lax.broadcasted_iota(jnp.int32, (ch, ch), 0)
    ib = lax.broadcasted_iota(jnp.int32, (ch, ch), 1)
    tri = (ia <= ib).astype(jnp.float32)
    lane = lax.broadcasted_iota(jnp.int32, (_ROWS, _NT), 1)
    valid = lane < (_NT - 1)
    for v, out in ((xv, syn_ref), (yv, obs_ref)):
        s = v - mind
        s_next = jnp.concatenate([s[1:, :], zrow], axis=0)
        tz = jnp.where(valid_t, (s + s_next) * 0.5, 0.0)
        carry = jnp.zeros((_ROWS, 1), jnp.float32)
        pieces = []
        for t in range(_NT // ch):
            blk = tz[t * ch : (t + 1) * ch, :]
            cs = (
                lax.dot_general(
                    blk,
                    tri,
                    dimension_numbers=(((0,), (0,)), ((), ())),
                    preferred_element_type=jnp.float32,
                )
                + carry
            )
            carry = cs[:, ch - 1 : ch]
            pieces.append(cs)
        c = jnp.concatenate(pieces, axis=1)
        total = jnp.sum(c, axis=1, keepdims=True) - carry
        out[...] = jnp.where(valid, c / total, 0.0)


def _tc_prep(x2d, y2d):
    grid = _NS // _ROWS
    in_spec = pl.BlockSpec((_NT, _ROWS), lambda i: (0, i))
    out_spec = pl.BlockSpec((_ROWS, _NT), lambda i: (i, 0))
    return pl.pallas_call(
        _tc_prep_body,
        grid=(grid,),
        in_specs=[in_spec, in_spec],
        out_specs=[out_spec, out_spec],
        out_shape=[
            jax.ShapeDtypeStruct((_NS, _NT), jnp.float32),
            jax.ShapeDtypeStruct((_NS, _NT), jnp.float32),
        ],
    )(x2d, y2d)


_UNROLL = 4


def _search_col(syn_v, obs_v, acc):
    lane = lax.iota(jnp.int32, 16)

    def chunk_body(k, a):
        for u in range(_UNROLL):
            ck = k * _UNROLL + u
            q = syn_v[pl.ds(ck * 16, 16)]
            pos = jnp.zeros((16,), jnp.int32)
            d = _NT // 2
            while d >= 1:
                probe = pos + (d - 1)
                v = plsc.load_gather(obs_v, [probe])
                pos = jnp.where(v < q, probe + 1, pos)
                d //= 2
            diff = (ck * 16 + 1 + lane - pos).astype(jnp.float32)
            a = a + diff * diff * q
        return a

    return lax.fori_loop(0, _NT // (16 * _UNROLL), chunk_body, acc)


def _sc_search_body(
    syn_hbm, obs_hbm, out_hbm, syn0, obs0, syn1, obs1, acc_v, sem0, sem1
):
    info = plsc.get_sparse_core_info()
    nc = info.num_cores
    wid = lax.axis_index("s") * nc + lax.axis_index("c")
    base = wid * _PER_W
    last = base + _PER_W - 1

    pltpu.sync_copy(syn_hbm.at[base], syn0)
    pltpu.sync_copy(obs_hbm.at[base], obs0)

    def col2_body(jj, acc):
        c0 = base + jj * 2
        nxt = jnp.minimum(c0 + 1, last)
        h1 = pltpu.async_copy(syn_hbm.at[nxt], syn1, sem1)
        h2 = pltpu.async_copy(obs_hbm.at[nxt], obs1, sem1)
        acc = _search_col(syn0, obs0, acc)
        h1.wait()
        h2.wait()
        nxt2 = jnp.minimum(c0 + 2, last)
        h3 = pltpu.async_copy(syn_hbm.at[nxt2], syn0, sem0)
        h4 = pltpu.async_copy(obs_hbm.at[nxt2], obs0, sem0)
        acc = _search_col(syn1, obs1, acc)
        h3.wait()
        h4.wait()
        return acc

    acc = lax.fori_loop(
        0, _PER_W // 2, col2_body, jnp.zeros((16,), jnp.float32)
    )
    acc_v[...] = acc
    pltpu.sync_copy(acc_v, out_hbm.at[wid])


def _sc_search(syn, obs):
    mesh = plsc.VectorSubcoreMesh(core_axis_name="c", subcore_axis_name="s")
    kern = functools.partial(
        pl.kernel,
        out_type=jax.ShapeDtypeStruct((_WORKERS, 16), jnp.float32),
        mesh=mesh,
        scratch_types=[
            pltpu.VMEM((_NT,), jnp.float32),
            pltpu.VMEM((_NT,), jnp.float32),
            pltpu.VMEM((_NT,), jnp.float32),
            pltpu.VMEM((_NT,), jnp.float32),
            pltpu.VMEM((16,), jnp.float32),
            pltpu.SemaphoreType.DMA,
            pltpu.SemaphoreType.DMA,
        ],
        compiler_params=pltpu.CompilerParams(needs_layout_passes=False),
    )(_sc_search_body)
    return kern(syn, obs)


def kernel(x, y):
    syn, obs = _tc_prep(x.reshape(_NT, -1), y.reshape(_NT, -1))
    part = _sc_search(syn, obs)
    return jnp.sum(part)
